# baseline (device time: 89311 ns/iter reference)
import jax
import jax.numpy as jnp
from jax import lax
from jax.experimental import pallas as pl
from jax.experimental.pallas import tpu as pltpu

N_DEV = 4
SCALE = 0.08838834764831843


def kernel(x, Wq, Wo, K_ext, V_ext, mode="full"):
    seq_per = x.shape[1]
    d_model = x.shape[2]
    skv = K_ext.shape[1]
    dh = K_ext.shape[3]
    h_per = Wq.shape[1] // dh
    hd_per = h_per * dh

    xb = x[0].astype(jnp.bfloat16)
    wq = Wq.astype(jnp.bfloat16)
    wo = Wo.astype(jnp.bfloat16)

    n_stage = 4

    def body(x_ref, wq_ref, wo_ref, k_any, v_any, out_ref,
             xg1, xg2, xg3, part0, part1, part2, part3,
             rsb1, rsb2, rsb3, attn_ref,
             kv_stage, kb_ref, vb_ref,
             kv_sems, ag_send, ag_recv, rs_send, rs_recv):
        xgs = [xg1, xg2, xg3]
        parts = [part0, part1, part2, part3]
        rsbs = [rsb1, rsb2, rsb3]
        my_pos = lax.axis_index("i")
        head0 = my_pos * h_per

        use_kv = mode not in ("nokv", "ringonly")
        use_attn = mode not in ("nocompute", "ringonly")
        use_comm = mode != "nocomm"

        def kv_head_dma(idx):
            src = k_any if idx % 2 == 0 else v_any
            h = idx // 2
            return pltpu.make_async_copy(
                src.at[0, :, head0 + h, :],
                kv_stage.at[idx % n_stage],
                kv_sems.at[idx % n_stage],
            )

        def kv_head_cast(idx):
            dst = kb_ref if idx % 2 == 0 else vb_ref
            h = idx // 2
            dst[:, h * dh:(h + 1) * dh] = (
                kv_stage[idx % n_stage].astype(jnp.bfloat16))

        kv_dmas = [None] * (2 * h_per)
        if use_kv:
            for idx in range(n_stage):
                kv_dmas[idx] = kv_head_dma(idx)
                kv_dmas[idx].start()

        if use_comm:
            barrier = pltpu.get_barrier_semaphore()
            for o in (1, 2, 3):
                pl.semaphore_signal(
                    barrier, inc=1,
                    device_id=(lax.rem(my_pos + o, N_DEV),),
                    device_id_type=pl.DeviceIdType.MESH,
                )
            pl.semaphore_wait(barrier, 3)

        ag = []
        if use_comm:
            for o in (1, 2, 3):
                r = pltpu.make_async_remote_copy(
                    src_ref=x_ref,
                    dst_ref=xgs[o - 1],
                    send_sem=ag_send.at[o - 1],
                    recv_sem=ag_recv.at[o - 1],
                    device_id=(lax.rem(my_pos + o, N_DEV),),
                    device_id_type=pl.DeviceIdType.MESH,
                )
                r.start()
                ag.append(r)

        def chunk_x(j):
            if j == 0 or not use_comm:
                return x_ref[...]
            return xgs[j - 1][...]

        def q_proj2(ja, jb):
            x2 = jnp.concatenate([chunk_x(ja), chunk_x(jb)], axis=0)
            q2 = jnp.dot(x2, wq_ref[...], preferred_element_type=jnp.float32)
            return (q2 * SCALE).astype(jnp.bfloat16)

        def head_attn(qj, h):
            qh = qj[:, h * dh:(h + 1) * dh]
            kh = kb_ref[:, h * dh:(h + 1) * dh]
            s = lax.dot_general(qh, kh, (((1,), (1,)), ((), ())),
                                preferred_element_type=jnp.float32)
            p = jnp.exp(s)
            l = jnp.sum(p, axis=1, keepdims=True)
            vh = vb_ref[:, h * dh:(h + 1) * dh]
            o = jnp.dot(p.astype(jnp.bfloat16), vh,
                        preferred_element_type=jnp.float32) / l
            attn_ref[:, h * dh:(h + 1) * dh] = o.astype(jnp.bfloat16)

        def compute_pair(ja, jb):
            if not use_attn:
                parts[ja][...] = chunk_x(ja)
                parts[jb][...] = chunk_x(jb)
                return
            q2 = q_proj2(ja, jb)
            for h in range(h_per):
                head_attn(q2, h)
            res = jnp.dot(attn_ref[...], wo_ref[...],
                          preferred_element_type=jnp.float32
                          ).astype(jnp.bfloat16)
            parts[ja][...] = res[:seq_per]
            parts[jb][...] = res[seq_per:]

        def rs_push(j):
            r = pltpu.make_async_remote_copy(
                src_ref=parts[j],
                dst_ref=rsbs[j - 1],
                send_sem=rs_send.at[j - 1],
                recv_sem=rs_recv.at[j - 1],
                device_id=(lax.rem(my_pos - j + N_DEV, N_DEV),),
                device_id_type=pl.DeviceIdType.MESH,
            )
            r.start()
            return r

        if use_comm:
            ag[0].wait()
            ag[1].wait()
        q12 = q_proj2(1, 2) if use_attn else None
        for h in range(h_per):
            if use_kv:
                for idx in (2 * h, 2 * h + 1):
                    kv_dmas[idx].wait()
                    kv_head_cast(idx)
                    if idx + n_stage < 2 * h_per:
                        kv_dmas[idx + n_stage] = kv_head_dma(idx + n_stage)
                        kv_dmas[idx + n_stage].start()
            if use_attn:
                head_attn(q12, h)
        if use_attn:
            res = jnp.dot(attn_ref[...], wo_ref[...],
                          preferred_element_type=jnp.float32
                          ).astype(jnp.bfloat16)
            parts[1][...] = res[:seq_per]
            parts[2][...] = res[seq_per:]
        else:
            parts[1][...] = chunk_x(1)
            parts[2][...] = chunk_x(2)
        rs = []
        if use_comm:
            rs.append(rs_push(1))
            rs.append(rs_push(2))
            ag[2].wait()

        compute_pair(3, 0)
        if use_comm:
            rs.append(rs_push(3))

        for r in rs:
            r.wait()
        if use_comm:
            out_ref[0] = (parts[0][...].astype(jnp.float32)
                          + rsbs[0][...].astype(jnp.float32)
                          + rsbs[1][...].astype(jnp.float32)
                          + rsbs[2][...].astype(jnp.float32))
        else:
            out_ref[0] = (parts[0][...].astype(jnp.float32)
                          + parts[1][...].astype(jnp.float32)
                          + parts[2][...].astype(jnp.float32)
                          + parts[3][...].astype(jnp.float32))

    return pl.pallas_call(
        body,
        out_shape=jax.ShapeDtypeStruct((1, seq_per, d_model), jnp.float32),
        in_specs=[
            pl.BlockSpec(memory_space=pltpu.VMEM),
            pl.BlockSpec(memory_space=pltpu.VMEM),
            pl.BlockSpec(memory_space=pltpu.VMEM),
            pl.BlockSpec(memory_space=pl.ANY),
            pl.BlockSpec(memory_space=pl.ANY),
        ],
        out_specs=pl.BlockSpec(memory_space=pltpu.VMEM),
        scratch_shapes=[
            pltpu.VMEM((seq_per, d_model), jnp.bfloat16),
            pltpu.VMEM((seq_per, d_model), jnp.bfloat16),
            pltpu.VMEM((seq_per, d_model), jnp.bfloat16),
            pltpu.VMEM((seq_per, d_model), jnp.bfloat16),
            pltpu.VMEM((seq_per, d_model), jnp.bfloat16),
            pltpu.VMEM((seq_per, d_model), jnp.bfloat16),
            pltpu.VMEM((seq_per, d_model), jnp.bfloat16),
            pltpu.VMEM((seq_per, d_model), jnp.bfloat16),
            pltpu.VMEM((seq_per, d_model), jnp.bfloat16),
            pltpu.VMEM((seq_per, d_model), jnp.bfloat16),
            pltpu.VMEM((2 * seq_per, d_model), jnp.bfloat16),
            pltpu.VMEM((n_stage, skv, dh), jnp.float32),
            pltpu.VMEM((skv, hd_per), jnp.bfloat16),
            pltpu.VMEM((skv, hd_per), jnp.bfloat16),
            pltpu.SemaphoreType.DMA((n_stage,)),
            pltpu.SemaphoreType.DMA((N_DEV - 1,)),
            pltpu.SemaphoreType.DMA((N_DEV - 1,)),
            pltpu.SemaphoreType.DMA((N_DEV - 1,)),
            pltpu.SemaphoreType.DMA((N_DEV - 1,)),
        ],
        compiler_params=pltpu.CompilerParams(
            collective_id=None if mode == "nocomm" else 0,
            vmem_limit_bytes=60 * 1024 * 1024,
        ),
    )(xb, wq, wo, K_ext, V_ext)


# device time: 75915 ns/iter; 1.1765x vs baseline; 1.1765x over previous
import jax
import jax.numpy as jnp
from jax import lax
from jax.experimental import pallas as pl
from jax.experimental.pallas import tpu as pltpu

N_DEV = 4
SCALE = 0.08838834764831843


def kernel(x, Wq, Wo, K_ext, V_ext, mode="full"):
    seq_per = x.shape[1]
    d_model = x.shape[2]
    skv = K_ext.shape[1]
    dh = K_ext.shape[3]
    h_per = Wq.shape[1] // dh
    hd_per = h_per * dh

    xb = x[0].astype(jnp.bfloat16)
    wq = Wq.astype(jnp.bfloat16)
    wo = Wo.astype(jnp.bfloat16)

    n_stage = 6

    def body(x_ref, wq_ref, wo_ref, k_any, v_any, out_ref,
             xg1, xg2, xg3, part0, part1, part2, part3,
             rsb1, rsb2, rsb3, attn_ref,
             kv_stage, kb_ref, vb_ref,
             kv_sems, ag_send, ag_recv, rs_send, rs_recv):
        xgs = [xg1, xg2, xg3]
        parts = [part0, part1, part2, part3]
        rsbs = [rsb1, rsb2, rsb3]
        my_pos = lax.axis_index("i")
        head0 = my_pos * h_per

        use_kv = mode not in ("nokv", "ringonly")
        use_attn = mode not in ("nocompute", "ringonly")
        use_comm = mode != "nocomm"

        def kv_head_dma(idx):
            src = k_any if idx % 2 == 0 else v_any
            h = idx // 2
            return pltpu.make_async_copy(
                src.at[0, :, head0 + h, :],
                kv_stage.at[idx % n_stage],
                kv_sems.at[idx % n_stage],
            )

        def kv_head_cast(idx):
            dst = kb_ref if idx % 2 == 0 else vb_ref
            h = idx // 2
            dst[:, h * dh:(h + 1) * dh] = (
                kv_stage[idx % n_stage].astype(jnp.bfloat16))

        kv_dmas = [None] * (2 * h_per)
        if use_kv:
            for idx in range(n_stage):
                kv_dmas[idx] = kv_head_dma(idx)
                kv_dmas[idx].start()

        if use_comm:
            barrier = pltpu.get_barrier_semaphore()
            for o in (1, 2, 3):
                pl.semaphore_signal(
                    barrier, inc=1,
                    device_id=(lax.rem(my_pos + o, N_DEV),),
                    device_id_type=pl.DeviceIdType.MESH,
                )
            pl.semaphore_wait(barrier, 3)

        ag = []
        if use_comm:
            for o in (1, 2, 3):
                r = pltpu.make_async_remote_copy(
                    src_ref=x_ref,
                    dst_ref=xgs[o - 1],
                    send_sem=ag_send.at[o - 1],
                    recv_sem=ag_recv.at[o - 1],
                    device_id=(lax.rem(my_pos + o, N_DEV),),
                    device_id_type=pl.DeviceIdType.MESH,
                )
                r.start()
                ag.append(r)

        def chunk_x(j):
            if j == 0 or not use_comm:
                return x_ref[...]
            return xgs[j - 1][...]

        def q_proj(j):
            qj = jnp.dot(chunk_x(j), wq_ref[...],
                         preferred_element_type=jnp.float32)
            return (qj * SCALE).astype(jnp.bfloat16)

        def head_attn(qj, h):
            qh = qj[:, h * dh:(h + 1) * dh]
            kh = kb_ref[:, h * dh:(h + 1) * dh]
            s = lax.dot_general(qh, kh, (((1,), (1,)), ((), ())),
                                preferred_element_type=jnp.float32)
            p = jnp.exp(s)
            l = jnp.sum(p, axis=1, keepdims=True)
            vh = vb_ref[:, h * dh:(h + 1) * dh]
            o = jnp.dot(p.astype(jnp.bfloat16), vh,
                        preferred_element_type=jnp.float32) / l
            attn_ref[:, h * dh:(h + 1) * dh] = o.astype(jnp.bfloat16)

        def out_proj(j):
            parts[j][...] = jnp.dot(attn_ref[...], wo_ref[...],
                                    preferred_element_type=jnp.float32
                                    ).astype(jnp.bfloat16)

        def compute_chunk(j):
            if not use_attn:
                parts[j][...] = chunk_x(j)
                return
            qj = q_proj(j)
            for h in range(h_per):
                head_attn(qj, h)
            out_proj(j)

        def rs_push(j):
            r = pltpu.make_async_remote_copy(
                src_ref=parts[j],
                dst_ref=rsbs[j - 1],
                send_sem=rs_send.at[j - 1],
                recv_sem=rs_recv.at[j - 1],
                device_id=(lax.rem(my_pos - j + N_DEV, N_DEV),),
                device_id_type=pl.DeviceIdType.MESH,
            )
            r.start()
            return r

        if use_comm:
            ag[0].wait()
        q1 = q_proj(1) if use_attn else None
        for h in range(h_per):
            if use_kv:
                for idx in (2 * h, 2 * h + 1):
                    kv_dmas[idx].wait()
                    kv_head_cast(idx)
                    if idx + n_stage < 2 * h_per:
                        kv_dmas[idx + n_stage] = kv_head_dma(idx + n_stage)
                        kv_dmas[idx + n_stage].start()
            if use_attn:
                head_attn(q1, h)
        if use_attn:
            out_proj(1)
        else:
            parts[1][...] = chunk_x(1)
        rs = [rs_push(1)] if use_comm else []

        for j in (2, 3):
            if use_comm:
                ag[j - 1].wait()
            compute_chunk(j)
            if use_comm:
                rs.append(rs_push(j))

        compute_chunk(0)

        for r in rs:
            r.wait()
        if use_comm:
            out_ref[0] = (parts[0][...].astype(jnp.float32)
                          + rsbs[0][...].astype(jnp.float32)
                          + rsbs[1][...].astype(jnp.float32)
                          + rsbs[2][...].astype(jnp.float32))
        else:
            out_ref[0] = (parts[0][...].astype(jnp.float32)
                          + parts[1][...].astype(jnp.float32)
                          + parts[2][...].astype(jnp.float32)
                          + parts[3][...].astype(jnp.float32))

    return pl.pallas_call(
        body,
        out_shape=jax.ShapeDtypeStruct((1, seq_per, d_model), jnp.float32),
        in_specs=[
            pl.BlockSpec(memory_space=pltpu.VMEM),
            pl.BlockSpec(memory_space=pltpu.VMEM),
            pl.BlockSpec(memory_space=pltpu.VMEM),
            pl.BlockSpec(memory_space=pl.ANY),
            pl.BlockSpec(memory_space=pl.ANY),
        ],
        out_specs=pl.BlockSpec(memory_space=pltpu.VMEM),
        scratch_shapes=[
            pltpu.VMEM((seq_per, d_model), jnp.bfloat16),
            pltpu.VMEM((seq_per, d_model), jnp.bfloat16),
            pltpu.VMEM((seq_per, d_model), jnp.bfloat16),
            pltpu.VMEM((seq_per, d_model), jnp.bfloat16),
            pltpu.VMEM((seq_per, d_model), jnp.bfloat16),
            pltpu.VMEM((seq_per, d_model), jnp.bfloat16),
            pltpu.VMEM((seq_per, d_model), jnp.bfloat16),
            pltpu.VMEM((seq_per, d_model), jnp.bfloat16),
            pltpu.VMEM((seq_per, d_model), jnp.bfloat16),
            pltpu.VMEM((seq_per, d_model), jnp.bfloat16),
            pltpu.VMEM((seq_per, d_model), jnp.bfloat16),
            pltpu.VMEM((n_stage, skv, dh), jnp.float32),
            pltpu.VMEM((skv, hd_per), jnp.bfloat16),
            pltpu.VMEM((skv, hd_per), jnp.bfloat16),
            pltpu.SemaphoreType.DMA((n_stage,)),
            pltpu.SemaphoreType.DMA((N_DEV - 1,)),
            pltpu.SemaphoreType.DMA((N_DEV - 1,)),
            pltpu.SemaphoreType.DMA((N_DEV - 1,)),
            pltpu.SemaphoreType.DMA((N_DEV - 1,)),
        ],
        compiler_params=pltpu.CompilerParams(
            collective_id=None if mode == "nocomm" else 0,
            vmem_limit_bytes=60 * 1024 * 1024,
        ),
    )(xb, wq, wo, K_ext, V_ext)
